# baseline (speedup 1.0000x reference)
import jax
import jax.numpy as jnp
from jax.experimental import pallas as pl

_BB = 128
_PB = 8


def _rpq_kernel(x_ref, w_ref, cb_ref, out_ref):
    t = jax.lax.dot_general(
        x_ref[...], w_ref[...],
        dimension_numbers=(((1,), (1,)), ((), ())),
        preferred_element_type=jnp.float32,
    )
    mu = jnp.mean(t, axis=1, keepdims=True)
    var = jnp.mean(jnp.square(t - mu), axis=1, keepdims=True)
    tn = (t - mu) * jax.lax.rsqrt(var + 1e-5)
    cb = cb_ref[...]
    cmu = jnp.mean(cb)
    cvar = jnp.mean(jnp.square(cb - cmu))
    cbn = (cb - cmu) * jax.lax.rsqrt(cvar + 1e-5)

    for i in range(_BB // _PB):
        chunk = tn[i * _PB:(i + 1) * _PB, :]
        d = chunk[:, None, :] - cbn[None, :, :]
        out_ref[i * _PB:(i + 1) * _PB, :] = jnp.argmin(d, axis=-1).astype(jnp.int32)


def kernel(input_values, W, code_book, raw_signal):
    B, D = input_values.shape
    Q = W.shape[0]
    return pl.pallas_call(
        _rpq_kernel,
        grid=(B // _BB,),
        in_specs=[
            pl.BlockSpec((_BB, D), lambda i: (i, 0)),
            pl.BlockSpec((Q, D), lambda i: (0, 0)),
            pl.BlockSpec((Q, Q), lambda i: (0, 0)),
        ],
        out_specs=pl.BlockSpec((_BB, Q), lambda i: (i, 0)),
        out_shape=jax.ShapeDtypeStruct((B, Q), jnp.int32),
    )(input_values, W, code_book)


# speedup vs baseline: 2.9248x; 2.9248x over previous
import jax
import jax.numpy as jnp
from jax.experimental import pallas as pl

_BB = 128
_PB = 8


def _rpq_kernel(x_ref, w_ref, cbt_ref, out_ref):
    t = jax.lax.dot_general(
        x_ref[...], w_ref[...],
        dimension_numbers=(((1,), (1,)), ((), ())),
        preferred_element_type=jnp.float32,
    )
    mu = jnp.mean(t, axis=1, keepdims=True)
    var = jnp.mean(jnp.square(t - mu), axis=1, keepdims=True)
    tts = t * jax.lax.rsqrt(var + 1e-5)
    cbt = cbt_ref[...]
    cmu = jnp.mean(cbt)
    cvar = jnp.mean(jnp.square(cbt - cmu))
    cbts = cbt * jax.lax.rsqrt(cvar + 1e-5)

    for i in range(_BB // _PB):
        chunk = tts[i * _PB:(i + 1) * _PB, :]
        d = chunk[:, :, None] - cbts[None, :, :]
        out_ref[i * _PB:(i + 1) * _PB, :] = jnp.argmin(d, axis=1).astype(jnp.int32)


def kernel(input_values, W, code_book, raw_signal):
    B, D = input_values.shape
    Q = W.shape[0]
    cbt = code_book[:Q].T
    return pl.pallas_call(
        _rpq_kernel,
        grid=(B // _BB,),
        in_specs=[
            pl.BlockSpec((_BB, D), lambda i: (i, 0)),
            pl.BlockSpec((Q, D), lambda i: (0, 0)),
            pl.BlockSpec((Q, Q), lambda i: (0, 0)),
        ],
        out_specs=pl.BlockSpec((_BB, Q), lambda i: (i, 0)),
        out_shape=jax.ShapeDtypeStruct((B, Q), jnp.int32),
    )(input_values, W, cbt)


# speedup vs baseline: 3.1751x; 1.0856x over previous
import jax
import jax.numpy as jnp
from jax.experimental import pallas as pl

_BB = 256
_PB = 8


def _rpq_kernel(x_ref, w_ref, cbt_ref, out_ref):
    t = jax.lax.dot_general(
        x_ref[...], w_ref[...],
        dimension_numbers=(((1,), (1,)), ((), ())),
        preferred_element_type=jnp.float32,
    )
    mu = jnp.mean(t, axis=1, keepdims=True)
    var = jnp.mean(jnp.square(t - mu), axis=1, keepdims=True)
    tts = t * jax.lax.rsqrt(var + 1e-5)
    cbt = cbt_ref[...]
    cmu = jnp.mean(cbt)
    cvar = jnp.mean(jnp.square(cbt - cmu))
    cbts = cbt * jax.lax.rsqrt(cvar + 1e-5)

    for i in range(_BB // _PB):
        chunk = tts[i * _PB:(i + 1) * _PB, :]
        d = chunk[:, :, None] - cbts[None, :, :]
        out_ref[i * _PB:(i + 1) * _PB, :] = jnp.argmin(d, axis=1).astype(jnp.int32)


def kernel(input_values, W, code_book, raw_signal):
    B, D = input_values.shape
    Q = W.shape[0]
    cbt = code_book[:Q].T
    return pl.pallas_call(
        _rpq_kernel,
        grid=(B // _BB,),
        in_specs=[
            pl.BlockSpec((_BB, D), lambda i: (i, 0)),
            pl.BlockSpec((Q, D), lambda i: (0, 0)),
            pl.BlockSpec((Q, Q), lambda i: (0, 0)),
        ],
        out_specs=pl.BlockSpec((_BB, Q), lambda i: (i, 0)),
        out_shape=jax.ShapeDtypeStruct((B, Q), jnp.int32),
    )(input_values, W, cbt)
